# baseline (speedup 1.0000x reference)
import functools

import jax
import jax.numpy as jnp
from jax import lax
from jax.experimental import pallas as pl
from jax.experimental.pallas import tpu as pltpu
from jax.experimental.pallas import tpu_sc as plsc

N = 10000
E = 320000
D = 128
D_OUT = 256
G = 64

NC = 2
NS = 16
NW = NC * NS
CHUNK = 128
CH = 80
CHR = 40
EPAD = NW * CH * CHUNK
NPAD = 10240
ROWS_PER_TILE = NPAD // NS
PAD_ROWS = NPAD - N

def _sc_agg_body(table_hbm, src_hbm, dst_hbm, out_hbm, src_v, dst_v, rows0,
                 rows1, acc, semg0, semg1, sems0, sems1):
    c = lax.axis_index("c")
    s = lax.axis_index("s")
    wid = c * NS + s

    zeros16 = jnp.zeros((16,), jnp.float32)

    def _zrow(i, _):
        for j in range(D // 16):
            rows0[i, pl.ds(j * 16, 16)] = zeros16
        return 0

    lax.fori_loop(0, CHUNK, _zrow, 0)
    for k in range(ROWS_PER_TILE // CHUNK):
        pltpu.sync_copy(
            rows0, acc.at[pl.ds(s * ROWS_PER_TILE + k * CHUNK, CHUNK)])
    plsc.subcore_barrier()

    for r in range(CH // CHR):
        pltpu.sync_copy(src_hbm.at[wid, pl.ds(r * CHR, CHR)], src_v)
        pltpu.sync_copy(dst_hbm.at[wid, pl.ds(r * CHR, CHR)], dst_v)
        pltpu.async_copy(table_hbm.at[pl.ds(0, CHUNK)], rows0, semg0)

        def _pair(i, _):
            a = 2 * i
            b = a + 1
            pltpu.async_copy(
                table_hbm.at[pl.ds((b % 70) * 128, CHUNK)], rows1, semg1)
            pltpu.make_async_copy(
                table_hbm.at[pl.ds(0, CHUNK)], rows0, semg0).wait()
            pltpu.sync_copy(rows0, acc.at[dst_v.at[a]], add=True)

            @pl.when(i < CHR // 2 - 1)
            def _():
                pltpu.async_copy(
                    table_hbm.at[pl.ds((a % 70) * 128, CHUNK)], rows0, semg0)

            pltpu.make_async_copy(
                table_hbm.at[pl.ds(0, CHUNK)], rows1, semg1).wait()
            pltpu.sync_copy(rows1, acc.at[dst_v.at[b]], add=True)
            return 0

        lax.fori_loop(0, CHR // 2, _pair, 0)
    plsc.subcore_barrier()

    pltpu.sync_copy(
        acc.at[pl.ds(s * ROWS_PER_TILE, ROWS_PER_TILE)],
        out_hbm.at[c, pl.ds(s * ROWS_PER_TILE, ROWS_PER_TILE)])


@functools.cache
def _build_sc_agg():
    mesh = plsc.VectorSubcoreMesh(core_axis_name="c", subcore_axis_name="s")
    return pl.kernel(
        _sc_agg_body,
        out_type=jax.ShapeDtypeStruct((NC, NPAD, D), jnp.float32),
        mesh=mesh,
        scratch_types=[
            pltpu.VMEM((CHR, CHUNK), jnp.int32),
            pltpu.VMEM((CHR, CHUNK), jnp.int32),
            pltpu.VMEM((CHUNK, D), jnp.float32),
            pltpu.VMEM((CHUNK, D), jnp.float32),
            pltpu.VMEM_SHARED((NPAD, D), jnp.float32),
            pltpu.SemaphoreType.DMA,
            pltpu.SemaphoreType.DMA,
            pltpu.SemaphoreType.DMA,
            pltpu.SemaphoreType.DMA,
        ],
    )


BR = 1000
NBLK = N // BR


def _mlp1_body(x_ref, p_ref, W1_ref, b1_ref, W2_ref, b2_ref, o_ref):
    a = x_ref[...] + p_ref[0] + p_ref[1]
    t = jnp.maximum(
        jnp.dot(a, W1_ref[...], preferred_element_type=jnp.float32)
        + b1_ref[...], 0.0)
    h = jnp.maximum(
        jnp.dot(t, W2_ref[...], preferred_element_type=jnp.float32)
        + b2_ref[...], 0.0)
    o_ref[...] = h


def _mlp2_body(h_ref, p_ref, W3_ref, b3_ref, W4_ref, b4_ref, batch_ref,
               gamma_ref, beta_ref, o_ref, sums, cnts):
    i = pl.program_id(0)
    a = h_ref[...] + p_ref[0] + p_ref[1]
    t = jnp.maximum(
        jnp.dot(a, W3_ref[...], preferred_element_type=jnp.float32)
        + b3_ref[...], 0.0)
    h2 = jnp.maximum(
        jnp.dot(t, W4_ref[...], preferred_element_type=jnp.float32)
        + b4_ref[...], 0.0)

    bb = batch_ref[0, 0, :]
    onehot = (lax.broadcasted_iota(jnp.int32, (G, BR), 0)
              == bb[None, :]).astype(jnp.float32)

    @pl.when(i == 0)
    def _():
        sums[...] = jnp.zeros_like(sums)
        cnts[...] = jnp.zeros_like(cnts)

    sums[...] += jnp.dot(onehot, h2, preferred_element_type=jnp.float32)
    cnts[...] += jnp.broadcast_to(
        jnp.sum(onehot, axis=1, keepdims=True), cnts.shape)

    @pl.when(i == NBLK - 1)
    def _():
        cnt = jnp.maximum(cnts[:, 0:1], 1.0)
        mean = sums[...] / cnt
        mu = jnp.mean(mean, axis=-1, keepdims=True)
        var = jnp.mean((mean - mu) ** 2, axis=-1, keepdims=True)
        o_ref[...] = ((mean - mu) * lax.rsqrt(var + 1e-5) * gamma_ref[...]
                      + beta_ref[...])


def _mlp1(x, partials, W1, b1, W2, b2):
    return pl.pallas_call(
        _mlp1_body,
        grid=(NBLK,),
        in_specs=[
            pl.BlockSpec((BR, D), lambda i: (i, 0)),
            pl.BlockSpec((NC, BR, D), lambda i: (0, i, 0)),
            pl.BlockSpec((D, D), lambda i: (0, 0)),
            pl.BlockSpec((1, D), lambda i: (0, 0)),
            pl.BlockSpec((D, D), lambda i: (0, 0)),
            pl.BlockSpec((1, D), lambda i: (0, 0)),
        ],
        out_specs=pl.BlockSpec((BR, D), lambda i: (i, 0)),
        out_shape=jax.ShapeDtypeStruct((N, D), jnp.float32),
    )(x, partials, W1, b1.reshape(1, D), W2, b2.reshape(1, D))


def _mlp2(h, partials, W3, b3, W4, b4, batch3, gamma, beta):
    return pl.pallas_call(
        _mlp2_body,
        grid=(NBLK,),
        in_specs=[
            pl.BlockSpec((BR, D), lambda i: (i, 0)),
            pl.BlockSpec((NC, BR, D), lambda i: (0, i, 0)),
            pl.BlockSpec((D, D), lambda i: (0, 0)),
            pl.BlockSpec((1, D), lambda i: (0, 0)),
            pl.BlockSpec((D, D_OUT), lambda i: (0, 0)),
            pl.BlockSpec((1, D_OUT), lambda i: (0, 0)),
            pl.BlockSpec((1, 1, BR), lambda i: (i, 0, 0)),
            pl.BlockSpec((1, D_OUT), lambda i: (0, 0)),
            pl.BlockSpec((1, D_OUT), lambda i: (0, 0)),
        ],
        out_specs=pl.BlockSpec((G, D_OUT), lambda i: (0, 0)),
        out_shape=jax.ShapeDtypeStruct((G, D_OUT), jnp.float32),
        scratch_shapes=[
            pltpu.VMEM((G, D_OUT), jnp.float32),
            pltpu.VMEM((G, D), jnp.float32),
        ],
    )(h, partials, W3, b3.reshape(1, D), W4, b4.reshape(1, D_OUT),
      batch3, gamma.reshape(1, D_OUT), beta.reshape(1, D_OUT))


def kernel(x, edge_index, batch, W1, b1, W2, b2, W3, b3, W4, b4, gamma, beta):
    src = edge_index[0]
    dst = edge_index[1]
    npad = EPAD - E
    pad_ar = jnp.arange(npad, dtype=jnp.int32)
    src_p = jnp.concatenate([src, pad_ar % N]).reshape(NW, CH, CHUNK)
    dst_p = jnp.concatenate([dst, N + pad_ar % PAD_ROWS]).reshape(NW, CH,
                                                                  CHUNK)
    batch3 = batch.reshape(NBLK, 1, BR)

    sc_agg = _build_sc_agg()
    partials1 = sc_agg(x, src_p, dst_p)
    h = _mlp1(x, partials1, W1, b1, W2, b2)
    partials2 = sc_agg(h, src_p, dst_p)
    return _mlp2(h, partials2, W3, b3, W4, b4, batch3, gamma, beta)


# speedup vs baseline: 1.1449x; 1.1449x over previous
import functools

import jax
import jax.numpy as jnp
from jax import lax
from jax.experimental import pallas as pl
from jax.experimental.pallas import tpu as pltpu
from jax.experimental.pallas import tpu_sc as plsc

N = 10000
E = 320000
D = 128
D_OUT = 256
G = 64

NC = 2
NS = 16
NW = NC * NS
CHUNK = 112
CH = 90
CHR = 18
NBUF = 3
EPAD = NW * CH * CHUNK
NPAD = 10112
ROWS_PER_TILE = NPAD // NS
PAD_ROWS = NPAD - N

def _sc_agg_body(table_hbm, src_hbm, dst_hbm, out_hbm, src_v, dst_v, rows0,
                 rows1, rows2, acc, semg0, semg1, semg2):
    c = lax.axis_index("c")
    s = lax.axis_index("s")
    wid = c * NS + s
    rows = (rows0, rows1, rows2)
    semg = (semg0, semg1, semg2)

    zeros16 = jnp.zeros((16,), jnp.float32)

    def _zrow(i, _):
        for j in range(D // 16):
            rows0[i, pl.ds(j * 16, 16)] = zeros16
        return 0

    lax.fori_loop(0, CHUNK, _zrow, 0)
    base = s * ROWS_PER_TILE
    nfull = ROWS_PER_TILE // CHUNK
    for k in range(nfull):
        pltpu.sync_copy(rows0, acc.at[pl.ds(base + k * CHUNK, CHUNK)])
    rem = ROWS_PER_TILE - nfull * CHUNK
    if rem:
        pltpu.sync_copy(rows0.at[pl.ds(0, rem)],
                        acc.at[pl.ds(base + nfull * CHUNK, rem)])
    plsc.subcore_barrier()

    for r in range(CH // CHR):
        pltpu.sync_copy(src_hbm.at[wid, r], src_v)
        pltpu.sync_copy(dst_hbm.at[wid, r], dst_v)
        pltpu.async_copy(table_hbm.at[src_v.at[0]], rows0, semg0)
        pltpu.async_copy(table_hbm.at[src_v.at[1]], rows1, semg1)

        def _triple(i, _):
            for k in range(NBUF):
                cidx = NBUF * i + k
                pltpu.make_async_copy(
                    table_hbm.at[src_v.at[cidx]], rows[k], semg[k]).wait()

                @pl.when(cidx + 2 < CHR)
                def _():
                    pltpu.async_copy(table_hbm.at[src_v.at[cidx + 2]],
                                     rows[(k + 2) % NBUF], semg[(k + 2) % NBUF])

                pltpu.sync_copy(rows[k], acc.at[dst_v.at[cidx]], add=True)
            return 0

        lax.fori_loop(0, CHR // NBUF, _triple, 0)
    plsc.subcore_barrier()

    pltpu.sync_copy(
        acc.at[pl.ds(s * ROWS_PER_TILE, ROWS_PER_TILE)],
        out_hbm.at[c, pl.ds(s * ROWS_PER_TILE, ROWS_PER_TILE)])


@functools.cache
def _build_sc_agg():
    mesh = plsc.VectorSubcoreMesh(core_axis_name="c", subcore_axis_name="s")
    return pl.kernel(
        _sc_agg_body,
        out_type=jax.ShapeDtypeStruct((NC, NPAD, D), jnp.float32),
        mesh=mesh,
        scratch_types=[
            pltpu.VMEM((CHR, CHUNK), jnp.int32),
            pltpu.VMEM((CHR, CHUNK), jnp.int32),
            pltpu.VMEM((CHUNK, D), jnp.float32),
            pltpu.VMEM((CHUNK, D), jnp.float32),
            pltpu.VMEM((CHUNK, D), jnp.float32),
            pltpu.VMEM_SHARED((NPAD, D), jnp.float32),
            pltpu.SemaphoreType.DMA,
            pltpu.SemaphoreType.DMA,
            pltpu.SemaphoreType.DMA,
        ],
    )


BR = 1000
NBLK = N // BR


def _mlp1_body(x_ref, p_ref, W1_ref, b1_ref, W2_ref, b2_ref, o_ref):
    a = x_ref[...] + p_ref[0] + p_ref[1]
    t = jnp.maximum(
        jnp.dot(a, W1_ref[...], preferred_element_type=jnp.float32)
        + b1_ref[...], 0.0)
    h = jnp.maximum(
        jnp.dot(t, W2_ref[...], preferred_element_type=jnp.float32)
        + b2_ref[...], 0.0)
    o_ref[...] = h


def _mlp2_body(h_ref, p_ref, W3_ref, b3_ref, W4_ref, b4_ref, batch_ref,
               gamma_ref, beta_ref, o_ref, sums, cnts):
    i = pl.program_id(0)
    a = h_ref[...] + p_ref[0] + p_ref[1]
    t = jnp.maximum(
        jnp.dot(a, W3_ref[...], preferred_element_type=jnp.float32)
        + b3_ref[...], 0.0)
    h2 = jnp.maximum(
        jnp.dot(t, W4_ref[...], preferred_element_type=jnp.float32)
        + b4_ref[...], 0.0)

    bb = batch_ref[0, 0, :]
    onehot = (lax.broadcasted_iota(jnp.int32, (G, BR), 0)
              == bb[None, :]).astype(jnp.float32)

    @pl.when(i == 0)
    def _():
        sums[...] = jnp.zeros_like(sums)
        cnts[...] = jnp.zeros_like(cnts)

    sums[...] += jnp.dot(onehot, h2, preferred_element_type=jnp.float32)
    cnts[...] += jnp.broadcast_to(
        jnp.sum(onehot, axis=1, keepdims=True), cnts.shape)

    @pl.when(i == NBLK - 1)
    def _():
        cnt = jnp.maximum(cnts[:, 0:1], 1.0)
        mean = sums[...] / cnt
        mu = jnp.mean(mean, axis=-1, keepdims=True)
        var = jnp.mean((mean - mu) ** 2, axis=-1, keepdims=True)
        o_ref[...] = ((mean - mu) * lax.rsqrt(var + 1e-5) * gamma_ref[...]
                      + beta_ref[...])


def _mlp1(x, partials, W1, b1, W2, b2):
    return pl.pallas_call(
        _mlp1_body,
        grid=(NBLK,),
        in_specs=[
            pl.BlockSpec((BR, D), lambda i: (i, 0)),
            pl.BlockSpec((NC, BR, D), lambda i: (0, i, 0)),
            pl.BlockSpec((D, D), lambda i: (0, 0)),
            pl.BlockSpec((1, D), lambda i: (0, 0)),
            pl.BlockSpec((D, D), lambda i: (0, 0)),
            pl.BlockSpec((1, D), lambda i: (0, 0)),
        ],
        out_specs=pl.BlockSpec((BR, D), lambda i: (i, 0)),
        out_shape=jax.ShapeDtypeStruct((N, D), jnp.float32),
    )(x, partials, W1, b1.reshape(1, D), W2, b2.reshape(1, D))


def _mlp2(h, partials, W3, b3, W4, b4, batch3, gamma, beta):
    return pl.pallas_call(
        _mlp2_body,
        grid=(NBLK,),
        in_specs=[
            pl.BlockSpec((BR, D), lambda i: (i, 0)),
            pl.BlockSpec((NC, BR, D), lambda i: (0, i, 0)),
            pl.BlockSpec((D, D), lambda i: (0, 0)),
            pl.BlockSpec((1, D), lambda i: (0, 0)),
            pl.BlockSpec((D, D_OUT), lambda i: (0, 0)),
            pl.BlockSpec((1, D_OUT), lambda i: (0, 0)),
            pl.BlockSpec((1, 1, BR), lambda i: (i, 0, 0)),
            pl.BlockSpec((1, D_OUT), lambda i: (0, 0)),
            pl.BlockSpec((1, D_OUT), lambda i: (0, 0)),
        ],
        out_specs=pl.BlockSpec((G, D_OUT), lambda i: (0, 0)),
        out_shape=jax.ShapeDtypeStruct((G, D_OUT), jnp.float32),
        scratch_shapes=[
            pltpu.VMEM((G, D_OUT), jnp.float32),
            pltpu.VMEM((G, D), jnp.float32),
        ],
    )(h, partials, W3, b3.reshape(1, D), W4, b4.reshape(1, D_OUT),
      batch3, gamma.reshape(1, D_OUT), beta.reshape(1, D_OUT))


def kernel(x, edge_index, batch, W1, b1, W2, b2, W3, b3, W4, b4, gamma, beta):
    src = edge_index[0]
    dst = edge_index[1]
    npad = EPAD - E
    pad_ar = jnp.arange(npad, dtype=jnp.int32)
    shape4 = (NW, CH // CHR, CHR, CHUNK)
    src_p = jnp.concatenate([src, pad_ar % N]).reshape(shape4)
    dst_p = jnp.concatenate([dst, N + pad_ar % PAD_ROWS]).reshape(shape4)
    batch3 = batch.reshape(NBLK, 1, BR)

    sc_agg = _build_sc_agg()
    partials1 = sc_agg(x, src_p, dst_p)
    h = _mlp1(x, partials1, W1, b1, W2, b2)
    partials2 = sc_agg(h, src_p, dst_p)
    return _mlp2(h, partials2, W3, b3, W4, b4, batch3, gamma, beta)


# speedup vs baseline: 1.1686x; 1.0207x over previous
import functools

import jax
import jax.numpy as jnp
from jax import lax
from jax.experimental import pallas as pl
from jax.experimental.pallas import tpu as pltpu
from jax.experimental.pallas import tpu_sc as plsc

N = 10000
E = 320000
D = 128
D_OUT = 256
G = 64

NC = 2
NS = 16
NW = NC * NS
CHUNK = 112
CH = 90
CHR = 18
NBUF = 3
EPAD = NW * CH * CHUNK
NPAD = 10112
ROWS_PER_TILE = NPAD // NS
PAD_ROWS = NPAD - N

def _sc_agg_body(table_hbm, src_hbm, dst_hbm, out_hbm, src_v, dst_v, rows0,
                 rows1, rows2, acc, semg0, semg1, semg2):
    c = lax.axis_index("c")
    s = lax.axis_index("s")
    wid = c * NS + s
    rows = (rows0, rows1, rows2)
    semg = (semg0, semg1, semg2)

    zeros16 = jnp.zeros((16,), jnp.float32)

    def _zrow(i, _):
        for j in range(D // 16):
            rows0[i, pl.ds(j * 16, 16)] = zeros16
        return 0

    lax.fori_loop(0, CHUNK, _zrow, 0)
    base = s * ROWS_PER_TILE
    nfull = ROWS_PER_TILE // CHUNK
    for k in range(nfull):
        pltpu.sync_copy(rows0, acc.at[pl.ds(base + k * CHUNK, CHUNK)])
    rem = ROWS_PER_TILE - nfull * CHUNK
    if rem:
        pltpu.sync_copy(rows0.at[pl.ds(0, rem)],
                        acc.at[pl.ds(base + nfull * CHUNK, rem)])
    plsc.subcore_barrier()

    for r in range(CH // CHR):
        pltpu.sync_copy(src_hbm.at[wid, r], src_v)
        pltpu.sync_copy(dst_hbm.at[wid, r], dst_v)
        pltpu.async_copy(table_hbm.at[src_v.at[0]], rows0, semg0)
        pltpu.async_copy(table_hbm.at[src_v.at[1]], rows1, semg1)

        def _triple(i, _):
            for k in range(NBUF):
                cidx = NBUF * i + k
                pltpu.make_async_copy(
                    table_hbm.at[src_v.at[cidx]], rows[k], semg[k]).wait()

                @pl.when(cidx + 2 < CHR)
                def _():
                    pltpu.async_copy(table_hbm.at[src_v.at[cidx + 2]],
                                     rows[(k + 2) % NBUF], semg[(k + 2) % NBUF])

                pltpu.sync_copy(rows[k], acc.at[dst_v.at[cidx]], add=True)
            return 0

        lax.fori_loop(0, CHR // NBUF, _triple, 0)
    plsc.subcore_barrier()

    pltpu.sync_copy(
        acc.at[pl.ds(s * ROWS_PER_TILE, ROWS_PER_TILE)],
        out_hbm.at[c, pl.ds(s * ROWS_PER_TILE, ROWS_PER_TILE)])


@functools.cache
def _build_sc_agg():
    mesh = plsc.VectorSubcoreMesh(core_axis_name="c", subcore_axis_name="s")
    return pl.kernel(
        _sc_agg_body,
        out_type=jax.ShapeDtypeStruct((NC, NPAD, D), jnp.float32),
        mesh=mesh,
        scratch_types=[
            pltpu.VMEM((CHR, CHUNK), jnp.int32),
            pltpu.VMEM((CHR, CHUNK), jnp.int32),
            pltpu.VMEM((CHUNK, D), jnp.float32),
            pltpu.VMEM((CHUNK, D), jnp.float32),
            pltpu.VMEM((CHUNK, D), jnp.float32),
            pltpu.VMEM_SHARED((NPAD, D), jnp.float32),
            pltpu.SemaphoreType.DMA,
            pltpu.SemaphoreType.DMA,
            pltpu.SemaphoreType.DMA,
        ],
    )


BR = 2000
NBLK = N // BR


def _mlp1_body(x_ref, p_ref, W1_ref, b1_ref, W2_ref, b2_ref, o_ref):
    a = x_ref[...] + p_ref[0] + p_ref[1]
    t = jnp.maximum(
        jnp.dot(a, W1_ref[...], preferred_element_type=jnp.float32,
                precision=lax.Precision.DEFAULT)
        + b1_ref[...], 0.0)
    h = jnp.maximum(
        jnp.dot(t, W2_ref[...], preferred_element_type=jnp.float32,
                precision=lax.Precision.DEFAULT)
        + b2_ref[...], 0.0)
    o_ref[...] = h


def _mlp2_body(h_ref, p_ref, W3_ref, b3_ref, W4_ref, b4_ref, batch_ref,
               gamma_ref, beta_ref, o_ref, sums, cnts):
    i = pl.program_id(0)
    a = h_ref[...] + p_ref[0] + p_ref[1]
    t = jnp.maximum(
        jnp.dot(a, W3_ref[...], preferred_element_type=jnp.float32,
                precision=lax.Precision.DEFAULT)
        + b3_ref[...], 0.0)
    h2 = jnp.maximum(
        jnp.dot(t, W4_ref[...], preferred_element_type=jnp.float32,
                precision=lax.Precision.DEFAULT)
        + b4_ref[...], 0.0)

    bb = batch_ref[0, 0, :]
    onehot = (lax.broadcasted_iota(jnp.int32, (G, BR), 0)
              == bb[None, :]).astype(jnp.float32)

    @pl.when(i == 0)
    def _():
        sums[...] = jnp.zeros_like(sums)
        cnts[...] = jnp.zeros_like(cnts)

    sums[...] += jnp.dot(onehot, h2, preferred_element_type=jnp.float32)
    cnts[...] += jnp.broadcast_to(
        jnp.sum(onehot, axis=1, keepdims=True), cnts.shape)

    @pl.when(i == NBLK - 1)
    def _():
        cnt = jnp.maximum(cnts[:, 0:1], 1.0)
        mean = sums[...] / cnt
        mu = jnp.mean(mean, axis=-1, keepdims=True)
        var = jnp.mean((mean - mu) ** 2, axis=-1, keepdims=True)
        o_ref[...] = ((mean - mu) * lax.rsqrt(var + 1e-5) * gamma_ref[...]
                      + beta_ref[...])


def _mlp1(x, partials, W1, b1, W2, b2):
    return pl.pallas_call(
        _mlp1_body,
        grid=(NBLK,),
        in_specs=[
            pl.BlockSpec((BR, D), lambda i: (i, 0)),
            pl.BlockSpec((NC, BR, D), lambda i: (0, i, 0)),
            pl.BlockSpec((D, D), lambda i: (0, 0)),
            pl.BlockSpec((1, D), lambda i: (0, 0)),
            pl.BlockSpec((D, D), lambda i: (0, 0)),
            pl.BlockSpec((1, D), lambda i: (0, 0)),
        ],
        out_specs=pl.BlockSpec((BR, D), lambda i: (i, 0)),
        out_shape=jax.ShapeDtypeStruct((N, D), jnp.float32),
    )(x, partials, W1, b1.reshape(1, D), W2, b2.reshape(1, D))


def _mlp2(h, partials, W3, b3, W4, b4, batch3, gamma, beta):
    return pl.pallas_call(
        _mlp2_body,
        grid=(NBLK,),
        in_specs=[
            pl.BlockSpec((BR, D), lambda i: (i, 0)),
            pl.BlockSpec((NC, BR, D), lambda i: (0, i, 0)),
            pl.BlockSpec((D, D), lambda i: (0, 0)),
            pl.BlockSpec((1, D), lambda i: (0, 0)),
            pl.BlockSpec((D, D_OUT), lambda i: (0, 0)),
            pl.BlockSpec((1, D_OUT), lambda i: (0, 0)),
            pl.BlockSpec((1, 1, BR), lambda i: (i, 0, 0)),
            pl.BlockSpec((1, D_OUT), lambda i: (0, 0)),
            pl.BlockSpec((1, D_OUT), lambda i: (0, 0)),
        ],
        out_specs=pl.BlockSpec((G, D_OUT), lambda i: (0, 0)),
        out_shape=jax.ShapeDtypeStruct((G, D_OUT), jnp.float32),
        scratch_shapes=[
            pltpu.VMEM((G, D_OUT), jnp.float32),
            pltpu.VMEM((G, D), jnp.float32),
        ],
    )(h, partials, W3, b3.reshape(1, D), W4, b4.reshape(1, D_OUT),
      batch3, gamma.reshape(1, D_OUT), beta.reshape(1, D_OUT))


def kernel(x, edge_index, batch, W1, b1, W2, b2, W3, b3, W4, b4, gamma, beta):
    src = edge_index[0]
    dst = edge_index[1]
    npad = EPAD - E
    pad_ar = jnp.arange(npad, dtype=jnp.int32)
    shape4 = (NW, CH // CHR, CHR, CHUNK)
    src_p = jnp.concatenate([src, pad_ar % N]).reshape(shape4)
    dst_p = jnp.concatenate([dst, N + pad_ar % PAD_ROWS]).reshape(shape4)
    batch3 = batch.reshape(NBLK, 1, BR)

    sc_agg = _build_sc_agg()
    partials1 = sc_agg(x, src_p, dst_p)
    h = _mlp1(x, partials1, W1, b1, W2, b2)
    partials2 = sc_agg(h, src_p, dst_p)
    return _mlp2(h, partials2, W3, b3, W4, b4, batch3, gamma, beta)


# speedup vs baseline: 1.1801x; 1.0099x over previous
import functools

import jax
import jax.numpy as jnp
from jax import lax
from jax.experimental import pallas as pl
from jax.experimental.pallas import tpu as pltpu
from jax.experimental.pallas import tpu_sc as plsc

N = 10000
E = 320000
D = 128
D_OUT = 256
G = 64

NC = 2
NS = 16
NW = NC * NS
CHUNK = 112
CH = 90
CHR = 18
NBUF = 3
EPAD = NW * CH * CHUNK
NPAD = 10112
ROWS_PER_TILE = NPAD // NS
PAD_ROWS = NPAD - N

def _sc_agg_body(table_hbm, src_hbm, dst_hbm, out_hbm, src_v, dst_v, rows0,
                 rows1, rows2, acc, semg0, semg1, semg2):
    c = lax.axis_index("c")
    s = lax.axis_index("s")
    wid = c * NS + s
    rows = (rows0, rows1, rows2)
    semg = (semg0, semg1, semg2)

    for r in range(CH // CHR):
        pltpu.sync_copy(src_hbm.at[wid, r], src_v)
        pltpu.sync_copy(dst_hbm.at[wid, r], dst_v)
        pltpu.async_copy(table_hbm.at[src_v.at[0]], rows0, semg0)
        pltpu.async_copy(table_hbm.at[src_v.at[1]], rows1, semg1)

        if r == 0:
            zeros16 = jnp.zeros((16,), jnp.float32)

            def _zrow(i, _):
                for j in range(D // 16):
                    rows2[i, pl.ds(j * 16, 16)] = zeros16
                return 0

            lax.fori_loop(0, CHUNK, _zrow, 0)
            base = s * ROWS_PER_TILE
            nfull = ROWS_PER_TILE // CHUNK
            for k in range(nfull):
                pltpu.sync_copy(rows2,
                                acc.at[pl.ds(base + k * CHUNK, CHUNK)])
            rem = ROWS_PER_TILE - nfull * CHUNK
            if rem:
                pltpu.sync_copy(rows2.at[pl.ds(0, rem)],
                                acc.at[pl.ds(base + nfull * CHUNK, rem)])
            plsc.subcore_barrier()

        def _triple(i, _):
            for k in range(NBUF):
                cidx = NBUF * i + k
                pltpu.make_async_copy(
                    table_hbm.at[src_v.at[cidx]], rows[k], semg[k]).wait()

                @pl.when(cidx + 2 < CHR)
                def _():
                    pltpu.async_copy(table_hbm.at[src_v.at[cidx + 2]],
                                     rows[(k + 2) % NBUF], semg[(k + 2) % NBUF])

                pltpu.sync_copy(rows[k], acc.at[dst_v.at[cidx]], add=True)
            return 0

        lax.fori_loop(0, CHR // NBUF, _triple, 0)
    plsc.subcore_barrier()

    pltpu.sync_copy(
        acc.at[pl.ds(s * ROWS_PER_TILE, ROWS_PER_TILE)],
        out_hbm.at[c, pl.ds(s * ROWS_PER_TILE, ROWS_PER_TILE)])


@functools.cache
def _build_sc_agg():
    mesh = plsc.VectorSubcoreMesh(core_axis_name="c", subcore_axis_name="s")
    return pl.kernel(
        _sc_agg_body,
        out_type=jax.ShapeDtypeStruct((NC, NPAD, D), jnp.float32),
        mesh=mesh,
        scratch_types=[
            pltpu.VMEM((CHR, CHUNK), jnp.int32),
            pltpu.VMEM((CHR, CHUNK), jnp.int32),
            pltpu.VMEM((CHUNK, D), jnp.float32),
            pltpu.VMEM((CHUNK, D), jnp.float32),
            pltpu.VMEM((CHUNK, D), jnp.float32),
            pltpu.VMEM_SHARED((NPAD, D), jnp.float32),
            pltpu.SemaphoreType.DMA,
            pltpu.SemaphoreType.DMA,
            pltpu.SemaphoreType.DMA,
        ],
    )


BR = 2000
NBLK = N // BR


def _mlp1_body(x_ref, p_ref, W1_ref, b1_ref, W2_ref, b2_ref, o_ref):
    a = x_ref[...] + p_ref[0] + p_ref[1]
    t = jnp.maximum(
        jnp.dot(a.astype(jnp.bfloat16), W1_ref[...].astype(jnp.bfloat16),
                preferred_element_type=jnp.float32)
        + b1_ref[...], 0.0)
    h = jnp.maximum(
        jnp.dot(t.astype(jnp.bfloat16), W2_ref[...].astype(jnp.bfloat16),
                preferred_element_type=jnp.float32)
        + b2_ref[...], 0.0)
    o_ref[...] = h


def _mlp2_body(h_ref, p_ref, W3_ref, b3_ref, W4_ref, b4_ref, batch_ref,
               gamma_ref, beta_ref, o_ref, sums, cnts):
    i = pl.program_id(0)
    a = h_ref[...] + p_ref[0] + p_ref[1]
    t = jnp.maximum(
        jnp.dot(a.astype(jnp.bfloat16), W3_ref[...].astype(jnp.bfloat16),
                preferred_element_type=jnp.float32)
        + b3_ref[...], 0.0)
    h2 = jnp.maximum(
        jnp.dot(t.astype(jnp.bfloat16), W4_ref[...].astype(jnp.bfloat16),
                preferred_element_type=jnp.float32)
        + b4_ref[...], 0.0)

    bb = batch_ref[0, 0, :]
    onehot = (lax.broadcasted_iota(jnp.int32, (G, BR), 0)
              == bb[None, :]).astype(jnp.float32)

    @pl.when(i == 0)
    def _():
        sums[...] = jnp.zeros_like(sums)
        cnts[...] = jnp.zeros_like(cnts)

    sums[...] += jnp.dot(onehot, h2, preferred_element_type=jnp.float32)
    cnts[...] += jnp.broadcast_to(
        jnp.sum(onehot, axis=1, keepdims=True), cnts.shape)

    @pl.when(i == NBLK - 1)
    def _():
        cnt = jnp.maximum(cnts[:, 0:1], 1.0)
        mean = sums[...] / cnt
        mu = jnp.mean(mean, axis=-1, keepdims=True)
        var = jnp.mean((mean - mu) ** 2, axis=-1, keepdims=True)
        o_ref[...] = ((mean - mu) * lax.rsqrt(var + 1e-5) * gamma_ref[...]
                      + beta_ref[...])


def _mlp1(x, partials, W1, b1, W2, b2):
    return pl.pallas_call(
        _mlp1_body,
        grid=(NBLK,),
        in_specs=[
            pl.BlockSpec((BR, D), lambda i: (i, 0)),
            pl.BlockSpec((NC, BR, D), lambda i: (0, i, 0)),
            pl.BlockSpec((D, D), lambda i: (0, 0)),
            pl.BlockSpec((1, D), lambda i: (0, 0)),
            pl.BlockSpec((D, D), lambda i: (0, 0)),
            pl.BlockSpec((1, D), lambda i: (0, 0)),
        ],
        out_specs=pl.BlockSpec((BR, D), lambda i: (i, 0)),
        out_shape=jax.ShapeDtypeStruct((N, D), jnp.float32),
    )(x, partials, W1, b1.reshape(1, D), W2, b2.reshape(1, D))


def _mlp2(h, partials, W3, b3, W4, b4, batch3, gamma, beta):
    return pl.pallas_call(
        _mlp2_body,
        grid=(NBLK,),
        in_specs=[
            pl.BlockSpec((BR, D), lambda i: (i, 0)),
            pl.BlockSpec((NC, BR, D), lambda i: (0, i, 0)),
            pl.BlockSpec((D, D), lambda i: (0, 0)),
            pl.BlockSpec((1, D), lambda i: (0, 0)),
            pl.BlockSpec((D, D_OUT), lambda i: (0, 0)),
            pl.BlockSpec((1, D_OUT), lambda i: (0, 0)),
            pl.BlockSpec((1, 1, BR), lambda i: (i, 0, 0)),
            pl.BlockSpec((1, D_OUT), lambda i: (0, 0)),
            pl.BlockSpec((1, D_OUT), lambda i: (0, 0)),
        ],
        out_specs=pl.BlockSpec((G, D_OUT), lambda i: (0, 0)),
        out_shape=jax.ShapeDtypeStruct((G, D_OUT), jnp.float32),
        scratch_shapes=[
            pltpu.VMEM((G, D_OUT), jnp.float32),
            pltpu.VMEM((G, D), jnp.float32),
        ],
    )(h, partials, W3, b3.reshape(1, D), W4, b4.reshape(1, D_OUT),
      batch3, gamma.reshape(1, D_OUT), beta.reshape(1, D_OUT))


def kernel(x, edge_index, batch, W1, b1, W2, b2, W3, b3, W4, b4, gamma, beta):
    src = edge_index[0]
    dst = edge_index[1]
    npad = EPAD - E
    pad_ar = jnp.arange(npad, dtype=jnp.int32)
    shape4 = (NW, CH // CHR, CHR, CHUNK)
    src_p = jnp.concatenate([src, pad_ar % N]).reshape(shape4)
    dst_p = jnp.concatenate([dst, N + pad_ar % PAD_ROWS]).reshape(shape4)
    batch3 = batch.reshape(NBLK, 1, BR)

    sc_agg = _build_sc_agg()
    partials1 = sc_agg(x, src_p, dst_p)
    h = _mlp1(x, partials1, W1, b1, W2, b2)
    partials2 = sc_agg(h, src_p, dst_p)
    return _mlp2(h, partials2, W3, b3, W4, b4, batch3, gamma, beta)


# speedup vs baseline: 1.2960x; 1.0981x over previous
import functools

import jax
import jax.numpy as jnp
from jax import lax
from jax.experimental import pallas as pl
from jax.experimental.pallas import tpu as pltpu
from jax.experimental.pallas import tpu_sc as plsc

N = 10000
E = 320000
D = 128
D_OUT = 256
G = 64

NC = 2
NS = 16
NW = NC * NS
CHUNK = 112
CH = 90
CHR = 6
NROUND = CH // CHR
NBUF = 3
EPAD = NW * CH * CHUNK
NPAD = 10112
ROWS_PER_TILE = NPAD // NS
PAD_ROWS = NPAD - N

def _sc_agg_body(table_hbm, src_hbm, dst_hbm, out_hbm, src_a, dst_a, src_b,
                 dst_b, rows0, rows1, rows2, acc, semg0, semg1, semg2, semi):
    c = lax.axis_index("c")
    s = lax.axis_index("s")
    wid = c * NS + s
    rows = (rows0, rows1, rows2)
    semg = (semg0, semg1, semg2)
    idx = ((src_a, dst_a), (src_b, dst_b))

    for r in range(NROUND):
        sv, dv = idx[r % 2]
        if r == 0:
            pltpu.sync_copy(src_hbm.at[wid, 0], sv)
            pltpu.sync_copy(dst_hbm.at[wid, 0], dv)
            pltpu.async_copy(table_hbm.at[sv.at[0]], rows0, semg0)
            pltpu.async_copy(table_hbm.at[sv.at[1]], rows1, semg1)

            zeros16 = jnp.zeros((16,), jnp.float32)

            def _zrow(i, _):
                for j in range(D // 16):
                    rows2[i, pl.ds(j * 16, 16)] = zeros16
                return 0

            lax.fori_loop(0, CHUNK, _zrow, 0)
            base = s * ROWS_PER_TILE
            nfull = ROWS_PER_TILE // CHUNK
            for k in range(nfull):
                pltpu.sync_copy(rows2,
                                acc.at[pl.ds(base + k * CHUNK, CHUNK)])
            rem = ROWS_PER_TILE - nfull * CHUNK
            if rem:
                pltpu.sync_copy(rows2.at[pl.ds(0, rem)],
                                acc.at[pl.ds(base + nfull * CHUNK, rem)])
            plsc.subcore_barrier()

        if r + 1 < NROUND:
            sv_n, dv_n = idx[(r + 1) % 2]
            pltpu.async_copy(src_hbm.at[wid, r + 1], sv_n, semi)
            pltpu.async_copy(dst_hbm.at[wid, r + 1], dv_n, semi)

        def _triple(i, _):
            for k in range(NBUF):
                cidx = NBUF * i + k
                pltpu.make_async_copy(
                    table_hbm.at[sv.at[cidx]], rows[k], semg[k]).wait()
                pltpu.async_copy(table_hbm.at[sv.at[cidx + 2]],
                                 rows[(k + 2) % NBUF], semg[(k + 2) % NBUF])
                pltpu.sync_copy(rows[k], acc.at[dv.at[cidx]], add=True)
            return 0

        lax.fori_loop(0, CHR // NBUF - 1, _triple, 0)

        for k in range(NBUF):
            cidx = CHR - NBUF + k
            pltpu.make_async_copy(
                table_hbm.at[sv.at[cidx]], rows[k], semg[k]).wait()
            nxt = cidx + 2
            if nxt < CHR:
                pltpu.async_copy(table_hbm.at[sv.at[nxt]],
                                 rows[(k + 2) % NBUF], semg[(k + 2) % NBUF])
            elif r + 1 < NROUND:
                if nxt == CHR:
                    sv_n, dv_n = idx[(r + 1) % 2]
                    pltpu.make_async_copy(
                        src_hbm.at[wid, r + 1], sv_n, semi).wait()
                    pltpu.make_async_copy(
                        dst_hbm.at[wid, r + 1], dv_n, semi).wait()
                sv_n = idx[(r + 1) % 2][0]
                pltpu.async_copy(table_hbm.at[sv_n.at[nxt - CHR]],
                                 rows[(k + 2) % NBUF], semg[(k + 2) % NBUF])
            pltpu.sync_copy(rows[k], acc.at[dv.at[cidx]], add=True)
    plsc.subcore_barrier()

    pltpu.sync_copy(
        acc.at[pl.ds(s * ROWS_PER_TILE, ROWS_PER_TILE)],
        out_hbm.at[c, pl.ds(s * ROWS_PER_TILE, ROWS_PER_TILE)])


@functools.cache
def _build_sc_agg():
    mesh = plsc.VectorSubcoreMesh(core_axis_name="c", subcore_axis_name="s")
    return pl.kernel(
        _sc_agg_body,
        out_type=jax.ShapeDtypeStruct((NC, NPAD, D), jnp.float32),
        mesh=mesh,
        scratch_types=[
            pltpu.VMEM((CHR, CHUNK), jnp.int32),
            pltpu.VMEM((CHR, CHUNK), jnp.int32),
            pltpu.VMEM((CHR, CHUNK), jnp.int32),
            pltpu.VMEM((CHR, CHUNK), jnp.int32),
            pltpu.VMEM((CHUNK, D), jnp.float32),
            pltpu.VMEM((CHUNK, D), jnp.float32),
            pltpu.VMEM((CHUNK, D), jnp.float32),
            pltpu.VMEM_SHARED((NPAD, D), jnp.float32),
            pltpu.SemaphoreType.DMA,
            pltpu.SemaphoreType.DMA,
            pltpu.SemaphoreType.DMA,
            pltpu.SemaphoreType.DMA,
        ],
    )


BR = 2000
NBLK = N // BR


def _mlp1_body(x_ref, p_ref, W1_ref, b1_ref, W2_ref, b2_ref, o_ref):
    a = x_ref[...] + p_ref[0] + p_ref[1]
    t = jnp.maximum(
        jnp.dot(a.astype(jnp.bfloat16), W1_ref[...].astype(jnp.bfloat16),
                preferred_element_type=jnp.float32)
        + b1_ref[...], 0.0)
    h = jnp.maximum(
        jnp.dot(t.astype(jnp.bfloat16), W2_ref[...].astype(jnp.bfloat16),
                preferred_element_type=jnp.float32)
        + b2_ref[...], 0.0)
    o_ref[...] = h


def _mlp2_body(h_ref, p_ref, W3_ref, b3_ref, W4_ref, b4_ref, batch_ref,
               gamma_ref, beta_ref, o_ref, sums, cnts):
    i = pl.program_id(0)
    a = h_ref[...] + p_ref[0] + p_ref[1]
    t = jnp.maximum(
        jnp.dot(a.astype(jnp.bfloat16), W3_ref[...].astype(jnp.bfloat16),
                preferred_element_type=jnp.float32)
        + b3_ref[...], 0.0)
    h2 = jnp.maximum(
        jnp.dot(t.astype(jnp.bfloat16), W4_ref[...].astype(jnp.bfloat16),
                preferred_element_type=jnp.float32)
        + b4_ref[...], 0.0)

    bb = batch_ref[0, 0, :]
    onehot = (lax.broadcasted_iota(jnp.int32, (G, BR), 0)
              == bb[None, :]).astype(jnp.float32)

    @pl.when(i == 0)
    def _():
        sums[...] = jnp.zeros_like(sums)
        cnts[...] = jnp.zeros_like(cnts)

    sums[...] += jnp.dot(onehot, h2, preferred_element_type=jnp.float32)
    cnts[...] += jnp.broadcast_to(
        jnp.sum(onehot, axis=1, keepdims=True), cnts.shape)

    @pl.when(i == NBLK - 1)
    def _():
        cnt = jnp.maximum(cnts[:, 0:1], 1.0)
        mean = sums[...] / cnt
        mu = jnp.mean(mean, axis=-1, keepdims=True)
        var = jnp.mean((mean - mu) ** 2, axis=-1, keepdims=True)
        o_ref[...] = ((mean - mu) * lax.rsqrt(var + 1e-5) * gamma_ref[...]
                      + beta_ref[...])


def _mlp1(x, partials, W1, b1, W2, b2):
    return pl.pallas_call(
        _mlp1_body,
        grid=(NBLK,),
        in_specs=[
            pl.BlockSpec((BR, D), lambda i: (i, 0)),
            pl.BlockSpec((NC, BR, D), lambda i: (0, i, 0)),
            pl.BlockSpec((D, D), lambda i: (0, 0)),
            pl.BlockSpec((1, D), lambda i: (0, 0)),
            pl.BlockSpec((D, D), lambda i: (0, 0)),
            pl.BlockSpec((1, D), lambda i: (0, 0)),
        ],
        out_specs=pl.BlockSpec((BR, D), lambda i: (i, 0)),
        out_shape=jax.ShapeDtypeStruct((N, D), jnp.float32),
    )(x, partials, W1, b1.reshape(1, D), W2, b2.reshape(1, D))


def _mlp2(h, partials, W3, b3, W4, b4, batch3, gamma, beta):
    return pl.pallas_call(
        _mlp2_body,
        grid=(NBLK,),
        in_specs=[
            pl.BlockSpec((BR, D), lambda i: (i, 0)),
            pl.BlockSpec((NC, BR, D), lambda i: (0, i, 0)),
            pl.BlockSpec((D, D), lambda i: (0, 0)),
            pl.BlockSpec((1, D), lambda i: (0, 0)),
            pl.BlockSpec((D, D_OUT), lambda i: (0, 0)),
            pl.BlockSpec((1, D_OUT), lambda i: (0, 0)),
            pl.BlockSpec((1, 1, BR), lambda i: (i, 0, 0)),
            pl.BlockSpec((1, D_OUT), lambda i: (0, 0)),
            pl.BlockSpec((1, D_OUT), lambda i: (0, 0)),
        ],
        out_specs=pl.BlockSpec((G, D_OUT), lambda i: (0, 0)),
        out_shape=jax.ShapeDtypeStruct((G, D_OUT), jnp.float32),
        scratch_shapes=[
            pltpu.VMEM((G, D_OUT), jnp.float32),
            pltpu.VMEM((G, D), jnp.float32),
        ],
    )(h, partials, W3, b3.reshape(1, D), W4, b4.reshape(1, D_OUT),
      batch3, gamma.reshape(1, D_OUT), beta.reshape(1, D_OUT))


def kernel(x, edge_index, batch, W1, b1, W2, b2, W3, b3, W4, b4, gamma, beta):
    src = edge_index[0]
    dst = edge_index[1]
    npad = EPAD - E
    pad_ar = jnp.arange(npad, dtype=jnp.int32)
    shape4 = (NW, CH // CHR, CHR, CHUNK)
    src_p = jnp.concatenate([src, pad_ar % N]).reshape(shape4)
    dst_p = jnp.concatenate([dst, N + pad_ar % PAD_ROWS]).reshape(shape4)
    batch3 = batch.reshape(NBLK, 1, BR)

    sc_agg = _build_sc_agg()
    partials1 = sc_agg(x, src_p, dst_p)
    h = _mlp1(x, partials1, W1, b1, W2, b2)
    partials2 = sc_agg(h, src_p, dst_p)
    return _mlp2(h, partials2, W3, b3, W4, b4, batch3, gamma, beta)
